# baseline (speedup 1.0000x reference)
import functools

import jax
import jax.numpy as jnp
from jax import lax
from jax.experimental import pallas as pl
from jax.experimental.pallas import tpu as pltpu
from jax.experimental.pallas import tpu_sc as plsc

B, S, D = 4, 8192, 1024


NW = 32
ROWS_PER_W = (B * S) // NW
R = 32
CHUNK = R * D
N_CHUNKS = ROWS_PER_W // R

_sc_mesh = plsc.VectorSubcoreMesh(core_axis_name="c", subcore_axis_name="s")


@functools.partial(
    pl.kernel,
    mesh=_sc_mesh,
    out_type=jax.ShapeDtypeStruct((B * S * D,), jnp.float32),
    scratch_types=[
        pltpu.VMEM((CHUNK,), jnp.float32),
        pltpu.VMEM((CHUNK,), jnp.float32),
    ],
)
def _sc_add(x_hbm, pe_hbm, out_hbm, x_v, pe_v):
    wid = lax.axis_index("s") * 2 + lax.axis_index("c")
    row0 = wid * ROWS_PER_W
    pe_row0 = lax.rem(row0, S)

    def chunk_body(ci, carry):
        xoff = (row0 + ci * R) * D
        poff = (pe_row0 + ci * R) * D
        pltpu.sync_copy(x_hbm.at[pl.ds(xoff, CHUNK)], x_v)
        pltpu.sync_copy(pe_hbm.at[pl.ds(poff, CHUNK)], pe_v)

        def add_body(k, c):
            sl = pl.ds(k * 16, 16)
            x_v[sl] = x_v[sl] + pe_v[sl]
            return c

        lax.fori_loop(0, CHUNK // 16, add_body, 0)
        pltpu.sync_copy(x_v, out_hbm.at[pl.ds(xoff, CHUNK)])
        return carry

    lax.fori_loop(0, N_CHUNKS, chunk_body, 0)


def kernel(x, pe):
    out = _sc_add(x.reshape(-1), pe.reshape(-1))
    return out.reshape(B, S, D)


# speedup vs baseline: 1.6390x; 1.6390x over previous
import functools

import jax
import jax.numpy as jnp
from jax import lax
from jax.experimental import pallas as pl
from jax.experimental.pallas import tpu as pltpu
from jax.experimental.pallas import tpu_sc as plsc

B, S, D = 4, 8192, 1024


NW = 32
PE_ROWS_W = S // NW
R = 8
RC = R * D
N_CHUNKS = PE_ROWS_W // R

_sc_mesh = plsc.VectorSubcoreMesh(core_axis_name="c", subcore_axis_name="s")


@functools.partial(
    pl.kernel,
    mesh=_sc_mesh,
    out_type=jax.ShapeDtypeStruct((B * S * D,), jnp.float32),
    scratch_types=[
        pltpu.VMEM((2, RC), jnp.float32),
        pltpu.VMEM((2, 4, RC), jnp.float32),
        pltpu.SemaphoreType.DMA((2,)),
        pltpu.SemaphoreType.DMA((2, 4)),
        pltpu.SemaphoreType.DMA((2, 4)),
    ],
)
def _sc_add(x_hbm, pe_hbm, out_hbm, pe_buf, x_buf, pe_sem, x_sem, out_sem):
    c_ax = lax.axis_index("c")
    s_ax = lax.axis_index("s")
    w = s_ax * 2 + c_ax
    pe_row0 = w * PE_ROWS_W

    def x_off(b, ci):
        return (b * S + pe_row0 + ci * R) * D

    def issue(ph, ci):
        pltpu.async_copy(
            pe_hbm.at[pl.ds((pe_row0 + ci * R) * D, RC)],
            pe_buf.at[ph],
            pe_sem.at[ph],
        )
        for b in range(4):
            pltpu.async_copy(
                x_hbm.at[pl.ds(x_off(b, ci), RC)],
                x_buf.at[ph, b],
                x_sem.at[ph, b],
            )

    issue(0, 0)

    def process(ph, ci):
        @pl.when(ci > 0)
        def _():
            for b in range(4):
                pltpu.make_async_copy(
                    x_buf.at[1 - ph, b], out_hbm.at[pl.ds(0, RC)], out_sem.at[1 - ph, b]
                ).wait()

        @pl.when(ci < N_CHUNKS - 1)
        def _():
            issue(1 - ph, ci + 1)

        pltpu.make_async_copy(
            pe_hbm.at[pl.ds(0, RC)], pe_buf.at[ph], pe_sem.at[ph]
        ).wait()
        for b in range(4):
            pltpu.make_async_copy(
                x_hbm.at[pl.ds(0, RC)], x_buf.at[ph, b], x_sem.at[ph, b]
            ).wait()

        def row_body(r, carry):
            base = r * D
            for h in range(2):
                hb = base + h * 512
                pe_vals = [pe_buf[ph, pl.ds(hb + k * 16, 16)] for k in range(32)]
                for b in range(4):
                    for k in range(32):
                        sl = pl.ds(hb + k * 16, 16)
                        x_buf[ph, b, sl] = x_buf[ph, b, sl] + pe_vals[k]
            return carry

        lax.fori_loop(0, R, row_body, 0)

        for b in range(4):
            pltpu.async_copy(
                x_buf.at[ph, b],
                out_hbm.at[pl.ds(x_off(b, ci), RC)],
                out_sem.at[ph, b],
            )

    def outer(c2, carry):
        for ph in range(2):
            process(ph, c2 * 2 + ph)
        return carry

    lax.fori_loop(0, N_CHUNKS // 2, outer, 0)

    for b in range(4):
        pltpu.make_async_copy(
            x_buf.at[1, b], out_hbm.at[pl.ds(0, RC)], out_sem.at[1, b]
        ).wait()


def kernel(x, pe):
    out = _sc_add(x.reshape(-1), pe.reshape(-1))
    return out.reshape(B, S, D)


# speedup vs baseline: 4.9249x; 3.0049x over previous
import functools

import jax
import jax.numpy as jnp
from jax import lax
from jax.experimental import pallas as pl
from jax.experimental.pallas import tpu as pltpu
from jax.experimental.pallas import tpu_sc as plsc

B, S, D = 4, 8192, 1024

NSUB = 16
ROWS_W = S // NSUB
R = 16
N_CHUNKS = ROWS_W // R

_sc_mesh = plsc.VectorSubcoreMesh(core_axis_name="c", subcore_axis_name="s")


@functools.partial(
    pl.kernel,
    mesh=_sc_mesh,
    out_type=jax.ShapeDtypeStruct((B, S, D), jnp.float32),
    scratch_types=[
        pltpu.VMEM((2, R, D), jnp.float32),
        pltpu.VMEM((2, 2, R, D), jnp.float32),
        pltpu.SemaphoreType.DMA((2,)),
        pltpu.SemaphoreType.DMA((2, 2)),
        pltpu.SemaphoreType.DMA((2, 2)),
    ],
)
def _sc_add(x_hbm, pe_hbm, out_hbm, pe_buf, x_buf, pe_sem, x_sem, out_sem):
    c_ax = lax.axis_index("c")
    s_ax = lax.axis_index("s")
    b0 = c_ax * 2
    row0 = s_ax * ROWS_W

    def issue(ph, ci):
        r = row0 + ci * R
        pltpu.async_copy(
            pe_hbm.at[pl.ds(r, R), :], pe_buf.at[ph], pe_sem.at[ph]
        )
        for j in range(2):
            pltpu.async_copy(
                x_hbm.at[b0 + j, pl.ds(r, R), :], x_buf.at[ph, j], x_sem.at[ph, j]
            )

    issue(0, 0)

    def process(ph, ci):
        @pl.when(ci > 0)
        def _():
            for j in range(2):
                pltpu.make_async_copy(
                    x_buf.at[1 - ph, j],
                    out_hbm.at[b0 + j, pl.ds(row0, R), :],
                    out_sem.at[1 - ph, j],
                ).wait()

        @pl.when(ci < N_CHUNKS - 1)
        def _():
            issue(1 - ph, ci + 1)

        pltpu.make_async_copy(
            pe_hbm.at[pl.ds(row0, R), :], pe_buf.at[ph], pe_sem.at[ph]
        ).wait()
        for j in range(2):
            pltpu.make_async_copy(
                x_hbm.at[b0 + j, pl.ds(row0, R), :], x_buf.at[ph, j], x_sem.at[ph, j]
            ).wait()

        def row_body(r, carry):
            for h in range(2):
                hb = h * 512
                pe_vals = [
                    pe_buf[ph, r, pl.ds(hb + k * 16, 16)] for k in range(32)
                ]
                for j in range(2):
                    for k in range(32):
                        sl = pl.ds(hb + k * 16, 16)
                        x_buf[ph, j, r, sl] = x_buf[ph, j, r, sl] + pe_vals[k]
            return carry

        lax.fori_loop(0, R, row_body, 0)

        r = row0 + ci * R
        for j in range(2):
            pltpu.async_copy(
                x_buf.at[ph, j],
                out_hbm.at[b0 + j, pl.ds(r, R), :],
                out_sem.at[ph, j],
            )

    def outer(c2, carry):
        for ph in range(2):
            process(ph, c2 * 2 + ph)
        return carry

    lax.fori_loop(0, N_CHUNKS // 2, outer, 0)

    for j in range(2):
        pltpu.make_async_copy(
            x_buf.at[1, j], out_hbm.at[b0 + j, pl.ds(row0, R), :], out_sem.at[1, j]
        ).wait()


def kernel(x, pe):
    return _sc_add(x, pe)


# speedup vs baseline: 5.8667x; 1.1912x over previous
import functools

import jax
import jax.numpy as jnp
from jax import lax
from jax.experimental import pallas as pl
from jax.experimental.pallas import tpu as pltpu
from jax.experimental.pallas import tpu_sc as plsc

B, S, D = 4, 8192, 1024

NW = 32
ROWS_W = S // NW
R = 8
N_CHUNKS = ROWS_W // R

_sc_mesh = plsc.VectorSubcoreMesh(core_axis_name="c", subcore_axis_name="s")


@functools.partial(
    pl.kernel,
    mesh=_sc_mesh,
    out_type=jax.ShapeDtypeStruct((B, S, D), jnp.float32),
    scratch_types=[
        pltpu.VMEM((2, R, D), jnp.float32),
        pltpu.VMEM((2, 4, R, D), jnp.float32),
        pltpu.SemaphoreType.DMA((2,)),
        pltpu.SemaphoreType.DMA((2, 4)),
        pltpu.SemaphoreType.DMA((2, 4)),
    ],
)
def _sc_add(x_hbm, pe_hbm, out_hbm, pe_buf, x_buf, pe_sem, x_sem, out_sem):
    c_ax = lax.axis_index("c")
    s_ax = lax.axis_index("s")
    w = s_ax * 2 + c_ax
    row0 = w * ROWS_W

    def issue(ph, ci):
        r = row0 + ci * R
        pltpu.async_copy(pe_hbm.at[pl.ds(r, R), :], pe_buf.at[ph], pe_sem.at[ph])
        for b in range(4):
            pltpu.async_copy(
                x_hbm.at[b, pl.ds(r, R), :], x_buf.at[ph, b], x_sem.at[ph, b]
            )

    issue(0, 0)

    def process(ph, ci):
        @pl.when(ci > 0)
        def _():
            for b in range(4):
                pltpu.make_async_copy(
                    x_buf.at[1 - ph, b],
                    out_hbm.at[b, pl.ds(row0, R), :],
                    out_sem.at[1 - ph, b],
                ).wait()

        @pl.when(ci < N_CHUNKS - 1)
        def _():
            issue(1 - ph, ci + 1)

        pltpu.make_async_copy(
            pe_hbm.at[pl.ds(row0, R), :], pe_buf.at[ph], pe_sem.at[ph]
        ).wait()
        for b in range(4):
            pltpu.make_async_copy(
                x_hbm.at[b, pl.ds(row0, R), :], x_buf.at[ph, b], x_sem.at[ph, b]
            ).wait()

        def row_body(r, carry):
            for h in range(2):
                hb = h * 512
                pe_vals = [
                    pe_buf[ph, r, pl.ds(hb + k * 16, 16)] for k in range(32)
                ]
                for b in range(4):
                    for k in range(32):
                        sl = pl.ds(hb + k * 16, 16)
                        x_buf[ph, b, r, sl] = x_buf[ph, b, r, sl] + pe_vals[k]
            return carry

        lax.fori_loop(0, R, row_body, 0)

        r = row0 + ci * R
        for b in range(4):
            pltpu.async_copy(
                x_buf.at[ph, b],
                out_hbm.at[b, pl.ds(r, R), :],
                out_sem.at[ph, b],
            )

    def outer(c2, carry):
        for ph in range(2):
            process(ph, c2 * 2 + ph)
        return carry

    lax.fori_loop(0, N_CHUNKS // 2, outer, 0)

    for b in range(4):
        pltpu.make_async_copy(
            x_buf.at[1, b], out_hbm.at[b, pl.ds(row0, R), :], out_sem.at[1, b]
        ).wait()


def kernel(x, pe):
    return _sc_add(x, pe)
